# Initial kernel scaffold; baseline (speedup 1.0000x reference)
import functools

import jax
import jax.numpy as jnp
from jax import lax
from jax.experimental import pallas as pl
from jax.experimental.pallas import tpu as pltpu
from jax.experimental.pallas import tpu_sc as plsc

_N = 10000
_E = 320000
_D = 128
_H = 128
_TD = 768
_OUT = 64

_NPAD = 10240
_NDUMMY = _NPAD - 1
_NW = 32
_CHUNK = 128
_NCH = 80
_EPT = _NCH * _CHUNK
_EPAD = _EPT * _NW
_RPT = _NPAD // 16

_BLK = 512
_NBLK = _NPAD // _BLK


def _sc_mesh():
    return plsc.VectorSubcoreMesh(core_axis_name="c", subcore_axis_name="s")


@functools.partial(
    pl.kernel,
    out_type=jax.ShapeDtypeStruct((2, _NPAD), jnp.float32),
    mesh=_sc_mesh(),
    scratch_types=[
        pltpu.VMEM((_NCH, _CHUNK), jnp.int32),
        pltpu.VMEM((_CHUNK,), jnp.float32),
        pltpu.VMEM((_RPT,), jnp.float32),
        pltpu.VMEM((_NPAD,), jnp.float32),
        pltpu.VMEM_SHARED((_NPAD,), jnp.float32),
    ],
)
def _deg_kernel(dst_hbm, deg_out, dstbuf, ones, zbuf, degbuf, sc_deg):
    cid = lax.axis_index("c")
    sid = lax.axis_index("s")
    wid = cid * 16 + sid
    z16 = jnp.zeros((16,), jnp.float32)
    o16 = jnp.ones((16,), jnp.float32)

    @pl.loop(0, _RPT // 16)
    def _(i):
        zbuf[pl.ds(i * 16, 16)] = z16

    for i in range(_CHUNK // 16):
        ones[pl.ds(i * 16, 16)] = o16

    pltpu.sync_copy(zbuf, sc_deg.at[pl.ds(sid * _RPT, _RPT)])
    pltpu.sync_copy(dst_hbm.at[wid], dstbuf)
    plsc.subcore_barrier()

    @pl.loop(0, _NCH)
    def _(j):
        pltpu.sync_copy(ones, sc_deg.at[dstbuf.at[j]], add=True)

    plsc.subcore_barrier()

    @pl.when(sid == 0)
    def _():
        pltpu.sync_copy(sc_deg, degbuf)
        pltpu.sync_copy(degbuf, deg_out.at[cid])


@functools.partial(
    pl.kernel,
    out_type=jax.ShapeDtypeStruct((2 * _NPAD, _D), jnp.float32),
    mesh=_sc_mesh(),
    scratch_types=[
        pltpu.VMEM((_NCH, _CHUNK), jnp.int32),
        pltpu.VMEM((_NCH, _CHUNK), jnp.int32),
        pltpu.VMEM((_CHUNK, _D), jnp.float32),
        pltpu.VMEM((_CHUNK, _D), jnp.float32),
        pltpu.VMEM((_RPT // 2, _D), jnp.float32),
        pltpu.SemaphoreType.DMA,
        pltpu.SemaphoreType.DMA,
        pltpu.VMEM_SHARED((_NPAD, _D), jnp.float32),
    ],
)
def _agg_kernel(y_hbm, src_hbm, dst_hbm, parts_out,
                srcbuf, dstbuf, rows0, rows1, obuf, sem0, sem1, acc):
    cid = lax.axis_index("c")
    sid = lax.axis_index("s")
    wid = cid * 16 + sid
    z16 = jnp.zeros((16,), jnp.float32)

    @pl.loop(0, _RPT // 2)
    def _(r):
        for c in range(_D // 16):
            obuf[r, pl.ds(c * 16, 16)] = z16

    pltpu.sync_copy(obuf, acc.at[pl.ds(sid * _RPT, _RPT // 2)])
    pltpu.sync_copy(obuf, acc.at[pl.ds(sid * _RPT + _RPT // 2, _RPT // 2)])
    pltpu.sync_copy(src_hbm.at[wid], srcbuf)
    pltpu.sync_copy(dst_hbm.at[wid], dstbuf)
    plsc.subcore_barrier()

    pltpu.async_copy(y_hbm.at[srcbuf.at[0]], rows0, sem0)

    @pl.loop(0, _NCH // 2)
    def _(t):
        j = t * 2
        pltpu.make_async_copy(y_hbm.at[srcbuf.at[j]], rows0, sem0).wait()
        pltpu.async_copy(y_hbm.at[srcbuf.at[j + 1]], rows1, sem1)
        pltpu.sync_copy(rows0, acc.at[dstbuf.at[j]], add=True)
        pltpu.make_async_copy(y_hbm.at[srcbuf.at[j + 1]], rows1, sem1).wait()
        jn = (j + 2) % _NCH
        pltpu.async_copy(y_hbm.at[srcbuf.at[jn]], rows0, sem0)
        pltpu.sync_copy(rows1, acc.at[dstbuf.at[j + 1]], add=True)

    pltpu.make_async_copy(y_hbm.at[srcbuf.at[0]], rows0, sem0).wait()
    plsc.subcore_barrier()

    base = cid * _NPAD + sid * _RPT
    pltpu.sync_copy(acc.at[pl.ds(sid * _RPT, _RPT // 2)], obuf)
    pltpu.sync_copy(obuf, parts_out.at[pl.ds(base, _RPT // 2)])
    pltpu.sync_copy(acc.at[pl.ds(sid * _RPT + _RPT // 2, _RPT // 2)], obuf)
    pltpu.sync_copy(obuf, parts_out.at[pl.ds(base + _RPT // 2, _RPT // 2)])


def _dinv_body(dp_ref, out_ref):
    p = dp_ref[...]
    deg = p[0:80, :] + p[80:160, :] + 1.0
    out_ref[...] = lax.rsqrt(deg)


_dinv_call = pl.pallas_call(
    _dinv_body,
    out_shape=jax.ShapeDtypeStruct((80, 128), jnp.float32),
)


def _y1_body(x_ref, dinv_ref, w_ref, out_ref):
    xw = jnp.dot(x_ref[...], w_ref[...],
                 preferred_element_type=jnp.float32,
                 precision=lax.Precision.HIGHEST)
    out_ref[...] = dinv_ref[...] * xw


_y1_call = pl.pallas_call(
    _y1_body,
    grid=(_NBLK,),
    in_specs=[
        pl.BlockSpec((_BLK, _D), lambda i: (i, 0)),
        pl.BlockSpec((_BLK, 1), lambda i: (i, 0)),
        pl.BlockSpec((_D, _H), lambda i: (0, 0)),
    ],
    out_specs=pl.BlockSpec((_BLK, _H), lambda i: (i, 0)),
    out_shape=jax.ShapeDtypeStruct((_NPAD, _H), jnp.float32),
)


def _layer_body(pa_ref, pb_ref, y_ref, dinv_ref, b_ref, w_ref, out_ref):
    dinv = dinv_ref[...]
    h = dinv * (pa_ref[...] + pb_ref[...] + y_ref[...]) + b_ref[...]
    h = jnp.maximum(h, 0.0)
    out_ref[...] = dinv * jnp.dot(h, w_ref[...],
                                  preferred_element_type=jnp.float32,
                                  precision=lax.Precision.HIGHEST)


_layer_call = pl.pallas_call(
    _layer_body,
    grid=(_NBLK,),
    in_specs=[
        pl.BlockSpec((_BLK, _H), lambda i: (i, 0)),
        pl.BlockSpec((_BLK, _H), lambda i: (i + _NBLK, 0)),
        pl.BlockSpec((_BLK, _H), lambda i: (i, 0)),
        pl.BlockSpec((_BLK, 1), lambda i: (i, 0)),
        pl.BlockSpec((1, _H), lambda i: (0, 0)),
        pl.BlockSpec((_H, _H), lambda i: (0, 0)),
    ],
    out_specs=pl.BlockSpec((_BLK, _H), lambda i: (i, 0)),
    out_shape=jax.ShapeDtypeStruct((_NPAD, _H), jnp.float32),
)


def _final_body(pa_ref, pb_ref, y_ref, dinv_ref, b_ref, wh_ref,
                txt_ref, wt_ref, bc_ref, out_ref):
    dinv = dinv_ref[...]
    h2 = dinv * (pa_ref[...] + pb_ref[...] + y_ref[...]) + b_ref[...]
    const = jnp.dot(txt_ref[...], wt_ref[...],
                    preferred_element_type=jnp.float32,
                    precision=lax.Precision.HIGHEST)
    out_ref[...] = (jnp.dot(h2, wh_ref[...],
                            preferred_element_type=jnp.float32,
                            precision=lax.Precision.HIGHEST)
                    + const + bc_ref[...])


_final_call = pl.pallas_call(
    _final_body,
    grid=(_NBLK,),
    in_specs=[
        pl.BlockSpec((_BLK, _H), lambda i: (i, 0)),
        pl.BlockSpec((_BLK, _H), lambda i: (i + _NBLK, 0)),
        pl.BlockSpec((_BLK, _H), lambda i: (i, 0)),
        pl.BlockSpec((_BLK, 1), lambda i: (i, 0)),
        pl.BlockSpec((1, _H), lambda i: (0, 0)),
        pl.BlockSpec((_H, _OUT), lambda i: (0, 0)),
        pl.BlockSpec((1, _TD), lambda i: (0, 0)),
        pl.BlockSpec((_TD, _OUT), lambda i: (0, 0)),
        pl.BlockSpec((1, _OUT), lambda i: (0, 0)),
    ],
    out_specs=pl.BlockSpec((_BLK, _OUT), lambda i: (i, 0)),
    out_shape=jax.ShapeDtypeStruct((_NPAD, _OUT), jnp.float32),
)


def kernel(x, edge_index, text_vec, W1, b1, W2, b2, Wc, bc):
    src = edge_index[0]
    dst = edge_index[1]
    pad = _EPAD - _E
    src_p = jnp.concatenate(
        [src, jnp.zeros((pad,), jnp.int32)]).reshape(_NW, _NCH, _CHUNK)
    dst_p = jnp.concatenate(
        [dst, jnp.full((pad,), _NDUMMY, jnp.int32)]).reshape(_NW, _NCH, _CHUNK)

    deg_parts = _deg_kernel(dst_p)
    dinv2d = _dinv_call(deg_parts.reshape(160, 128))
    dinv_col = dinv2d.reshape(_NPAD, 1)

    x_p = jnp.pad(x, ((0, _NPAD - _N), (0, 0)))
    y1 = _y1_call(x_p, dinv_col, W1)
    p1 = _agg_kernel(y1, src_p, dst_p)
    y2 = _layer_call(p1, p1, y1, dinv_col, b1.reshape(1, _H), W2)
    p2 = _agg_kernel(y2, src_p, dst_p)
    logits = _final_call(p2, p2, y2, dinv_col, b2.reshape(1, _H),
                         Wc[:, :_H].T, text_vec.reshape(1, _TD),
                         Wc[:, _H:].T, bc.reshape(1, _OUT))
    return logits[:_N]


# speedup vs baseline: 8.0757x; 8.0757x over previous
import functools

import jax
import jax.numpy as jnp
from jax import lax
from jax.experimental import pallas as pl
from jax.experimental.pallas import tpu as pltpu
from jax.experimental.pallas import tpu_sc as plsc

_N = 10000
_E = 320000
_D = 128
_H = 128
_TD = 768
_OUT = 64

_NPAD = 10240
_NDUMMY = _NPAD - 1
_NW = 32
_CHUNK = 128
_NCH = 80
_EPT = _NCH * _CHUNK
_EPAD = _EPT * _NW
_RPT = _NPAD // 16

_BLK = 512
_NBLK = _NPAD // _BLK


def _sc_mesh():
    return plsc.VectorSubcoreMesh(core_axis_name="c", subcore_axis_name="s")


@functools.partial(
    pl.kernel,
    out_type=jax.ShapeDtypeStruct((2, _NPAD), jnp.float32),
    mesh=_sc_mesh(),
    scratch_types=[
        pltpu.VMEM((_NCH, _CHUNK), jnp.int32),
        pltpu.VMEM((_CHUNK,), jnp.float32),
        pltpu.VMEM((_RPT,), jnp.float32),
        pltpu.VMEM((_NPAD,), jnp.float32),
        pltpu.VMEM_SHARED((_NPAD,), jnp.float32),
    ],
)
def _deg_kernel(dst_hbm, deg_out, dstbuf, ones, zbuf, degbuf, sc_deg):
    cid = lax.axis_index("c")
    sid = lax.axis_index("s")
    wid = cid * 16 + sid
    z16 = jnp.zeros((16,), jnp.float32)
    o16 = jnp.ones((16,), jnp.float32)

    @pl.loop(0, _RPT // 16)
    def _(i):
        zbuf[pl.ds(i * 16, 16)] = z16

    for i in range(_CHUNK // 16):
        ones[pl.ds(i * 16, 16)] = o16

    pltpu.sync_copy(zbuf, sc_deg.at[pl.ds(sid * _RPT, _RPT)])
    pltpu.sync_copy(dst_hbm.at[wid], dstbuf)
    plsc.subcore_barrier()

    @pl.loop(0, _NCH)
    def _(j):
        pltpu.sync_copy(ones, sc_deg.at[dstbuf.at[j]], add=True)

    plsc.subcore_barrier()

    @pl.when(sid == 0)
    def _():
        pltpu.sync_copy(sc_deg, degbuf)
        pltpu.sync_copy(degbuf, deg_out.at[cid])


_GRP = 16
_NGRP = _NCH // _GRP
_OB = 64


@functools.partial(
    pl.kernel,
    out_type=jax.ShapeDtypeStruct((2 * _NPAD, _D), jnp.float32),
    mesh=_sc_mesh(),
    scratch_types=[
        pltpu.VMEM((_GRP, _CHUNK), jnp.int32),
        pltpu.VMEM((_GRP, _CHUNK), jnp.int32),
        pltpu.VMEM((_CHUNK, _D), jnp.float32),
        pltpu.VMEM((_CHUNK, _D), jnp.float32),
        pltpu.VMEM((_OB, _D), jnp.float32),
        pltpu.SemaphoreType.DMA,
        pltpu.SemaphoreType.DMA,
        pltpu.VMEM_SHARED((_NPAD, _D), jnp.float32),
    ],
)
def _agg_kernel(y_hbm, src_hbm, dst_hbm, parts_out,
                sgrp, dgrp, rows0, rows1, obuf, sem0, sem1, acc):
    cid = lax.axis_index("c")
    sid = lax.axis_index("s")
    wid = cid * 16 + sid
    z16 = jnp.zeros((16,), jnp.float32)

    @pl.loop(0, _OB)
    def _(r):
        for c in range(_D // 16):
            obuf[r, pl.ds(c * 16, 16)] = z16

    @pl.loop(0, _RPT // _OB)
    def _(k):
        pltpu.sync_copy(obuf, acc.at[pl.ds(sid * _RPT + k * _OB, _OB)])

    plsc.subcore_barrier()

    @pl.loop(0, _NGRP)
    def _(g):
        pltpu.sync_copy(src_hbm.at[wid, pl.ds(g * _GRP, _GRP)], sgrp)
        pltpu.sync_copy(dst_hbm.at[wid, pl.ds(g * _GRP, _GRP)], dgrp)
        pltpu.async_copy(y_hbm.at[sgrp.at[0]], rows0, sem0)

        @pl.loop(0, _GRP // 2)
        def _(t):
            j = t * 2
            pltpu.make_async_copy(y_hbm.at[sgrp.at[j]], rows0, sem0).wait()
            pltpu.async_copy(y_hbm.at[sgrp.at[j + 1]], rows1, sem1)
            pltpu.sync_copy(rows0, acc.at[dgrp.at[j]], add=True)
            pltpu.make_async_copy(y_hbm.at[sgrp.at[j + 1]], rows1, sem1).wait()
            jn = (j + 2) % _GRP
            pltpu.async_copy(y_hbm.at[sgrp.at[jn]], rows0, sem0)
            pltpu.sync_copy(rows1, acc.at[dgrp.at[j + 1]], add=True)

        pltpu.make_async_copy(y_hbm.at[sgrp.at[0]], rows0, sem0).wait()

    plsc.subcore_barrier()

    @pl.loop(0, _RPT // _OB)
    def _(k):
        pltpu.sync_copy(acc.at[pl.ds(sid * _RPT + k * _OB, _OB)], obuf)
        pltpu.sync_copy(
            obuf,
            parts_out.at[pl.ds(cid * _NPAD + sid * _RPT + k * _OB, _OB)])


def _dinv_body(dp_ref, out_ref):
    p = dp_ref[...]
    deg = p[0:80, :] + p[80:160, :] + 1.0
    out_ref[...] = lax.rsqrt(deg)


_dinv_call = pl.pallas_call(
    _dinv_body,
    out_shape=jax.ShapeDtypeStruct((80, 128), jnp.float32),
)


def _y1_body(x_ref, dinv_ref, w_ref, out_ref):
    xw = jnp.dot(x_ref[...], w_ref[...],
                 preferred_element_type=jnp.float32,
                 precision=lax.Precision.HIGHEST)
    out_ref[...] = dinv_ref[...] * xw


_y1_call = pl.pallas_call(
    _y1_body,
    grid=(_NBLK,),
    in_specs=[
        pl.BlockSpec((_BLK, _D), lambda i: (i, 0)),
        pl.BlockSpec((_BLK, 1), lambda i: (i, 0)),
        pl.BlockSpec((_D, _H), lambda i: (0, 0)),
    ],
    out_specs=pl.BlockSpec((_BLK, _H), lambda i: (i, 0)),
    out_shape=jax.ShapeDtypeStruct((_NPAD, _H), jnp.float32),
)


def _layer_body(pa_ref, pb_ref, y_ref, dinv_ref, b_ref, w_ref, out_ref):
    dinv = dinv_ref[...]
    h = dinv * (pa_ref[...] + pb_ref[...] + y_ref[...]) + b_ref[...]
    h = jnp.maximum(h, 0.0)
    out_ref[...] = dinv * jnp.dot(h, w_ref[...],
                                  preferred_element_type=jnp.float32,
                                  precision=lax.Precision.HIGHEST)


_layer_call = pl.pallas_call(
    _layer_body,
    grid=(_NBLK,),
    in_specs=[
        pl.BlockSpec((_BLK, _H), lambda i: (i, 0)),
        pl.BlockSpec((_BLK, _H), lambda i: (i + _NBLK, 0)),
        pl.BlockSpec((_BLK, _H), lambda i: (i, 0)),
        pl.BlockSpec((_BLK, 1), lambda i: (i, 0)),
        pl.BlockSpec((1, _H), lambda i: (0, 0)),
        pl.BlockSpec((_H, _H), lambda i: (0, 0)),
    ],
    out_specs=pl.BlockSpec((_BLK, _H), lambda i: (i, 0)),
    out_shape=jax.ShapeDtypeStruct((_NPAD, _H), jnp.float32),
)


def _final_body(pa_ref, pb_ref, y_ref, dinv_ref, b_ref, wh_ref,
                txt_ref, wt_ref, bc_ref, out_ref):
    dinv = dinv_ref[...]
    h2 = dinv * (pa_ref[...] + pb_ref[...] + y_ref[...]) + b_ref[...]
    const = jnp.dot(txt_ref[...], wt_ref[...],
                    preferred_element_type=jnp.float32,
                    precision=lax.Precision.HIGHEST)
    out_ref[...] = (jnp.dot(h2, wh_ref[...],
                            preferred_element_type=jnp.float32,
                            precision=lax.Precision.HIGHEST)
                    + const + bc_ref[...])


_final_call = pl.pallas_call(
    _final_body,
    grid=(_NBLK,),
    in_specs=[
        pl.BlockSpec((_BLK, _H), lambda i: (i, 0)),
        pl.BlockSpec((_BLK, _H), lambda i: (i + _NBLK, 0)),
        pl.BlockSpec((_BLK, _H), lambda i: (i, 0)),
        pl.BlockSpec((_BLK, 1), lambda i: (i, 0)),
        pl.BlockSpec((1, _H), lambda i: (0, 0)),
        pl.BlockSpec((_H, _OUT), lambda i: (0, 0)),
        pl.BlockSpec((1, _TD), lambda i: (0, 0)),
        pl.BlockSpec((_TD, _OUT), lambda i: (0, 0)),
        pl.BlockSpec((1, _OUT), lambda i: (0, 0)),
    ],
    out_specs=pl.BlockSpec((_BLK, _OUT), lambda i: (i, 0)),
    out_shape=jax.ShapeDtypeStruct((_NPAD, _OUT), jnp.float32),
)


def kernel(x, edge_index, text_vec, W1, b1, W2, b2, Wc, bc):
    src = edge_index[0]
    dst = edge_index[1]
    pad = _EPAD - _E
    src_p = jnp.concatenate(
        [src, jnp.zeros((pad,), jnp.int32)]).reshape(_NW, _NCH, _CHUNK)
    dst_p = jnp.concatenate(
        [dst, jnp.full((pad,), _NDUMMY, jnp.int32)]).reshape(_NW, _NCH, _CHUNK)

    deg_parts = _deg_kernel(dst_p)
    dinv2d = _dinv_call(deg_parts.reshape(160, 128))
    dinv_col = dinv2d.reshape(_NPAD, 1)

    x_p = jnp.pad(x, ((0, _NPAD - _N), (0, 0)))
    y1 = _y1_call(x_p, dinv_col, W1)
    p1 = _agg_kernel(y1, src_p, dst_p)
    y2 = _layer_call(p1, p1, y1, dinv_col, b1.reshape(1, _H), W2)
    p2 = _agg_kernel(y2, src_p, dst_p)
    logits = _final_call(p2, p2, y2, dinv_col, b2.reshape(1, _H),
                         Wc[:, :_H].T, text_vec.reshape(1, _TD),
                         Wc[:, _H:].T, bc.reshape(1, _OUT))
    return logits[:_N]


# speedup vs baseline: 25.0536x; 3.1023x over previous
import functools

import jax
import jax.numpy as jnp
from jax import lax
from jax.experimental import pallas as pl
from jax.experimental.pallas import tpu as pltpu
from jax.experimental.pallas import tpu_sc as plsc

_N = 10000
_E = 320000
_D = 128
_H = 128
_TD = 768
_OUT = 64

_NPAD = 10240
_NDUMMY = _NPAD - 1
_NW = 32
_CHUNK = 128
_NCH = 80
_EPT = _NCH * _CHUNK
_EPAD = _EPT * _NW
_RPT = _NPAD // 16

_BLK = 512
_NBLK = _NPAD // _BLK


def _sc_mesh():
    return plsc.VectorSubcoreMesh(core_axis_name="c", subcore_axis_name="s")


@functools.partial(
    pl.kernel,
    out_type=jax.ShapeDtypeStruct((2, _NPAD), jnp.float32),
    mesh=_sc_mesh(),
    scratch_types=[
        pltpu.VMEM((_NCH, _CHUNK), jnp.int32),
        pltpu.VMEM((_CHUNK,), jnp.float32),
        pltpu.VMEM((_RPT,), jnp.float32),
        pltpu.VMEM((_NPAD,), jnp.float32),
        pltpu.VMEM_SHARED((_NPAD,), jnp.float32),
    ],
)
def _deg_kernel(dst_hbm, deg_out, dstbuf, ones, zbuf, degbuf, sc_deg):
    cid = lax.axis_index("c")
    sid = lax.axis_index("s")
    wid = cid * 16 + sid
    z16 = jnp.zeros((16,), jnp.float32)
    o16 = jnp.ones((16,), jnp.float32)

    @pl.loop(0, _RPT // 16)
    def _(i):
        zbuf[pl.ds(i * 16, 16)] = z16

    for i in range(_CHUNK // 16):
        ones[pl.ds(i * 16, 16)] = o16

    pltpu.sync_copy(zbuf, sc_deg.at[pl.ds(sid * _RPT, _RPT)])
    pltpu.sync_copy(dst_hbm.at[wid], dstbuf)
    plsc.subcore_barrier()

    @pl.loop(0, _NCH)
    def _(j):
        pltpu.sync_copy(ones, sc_deg.at[dstbuf.at[j]], add=True)

    plsc.subcore_barrier()

    @pl.when(sid == 0)
    def _():
        pltpu.sync_copy(sc_deg, degbuf)
        pltpu.sync_copy(degbuf, deg_out.at[cid])


_GRP = 16
_NGRP = _NCH // _GRP
_OB = 64


@functools.partial(
    pl.kernel,
    out_type=jax.ShapeDtypeStruct((2 * _NPAD, _D), jnp.float32),
    mesh=_sc_mesh(),
    scratch_types=[
        pltpu.VMEM((_GRP, _CHUNK), jnp.int32),
        pltpu.VMEM((_GRP, _CHUNK), jnp.int32),
        pltpu.VMEM((_CHUNK, _D), jnp.float32),
        pltpu.VMEM((_CHUNK, _D), jnp.float32),
        pltpu.VMEM((_OB, _D), jnp.float32),
        pltpu.SemaphoreType.DMA,
        pltpu.SemaphoreType.DMA,
        pltpu.VMEM_SHARED((_NPAD, _D), jnp.float32),
    ],
)
def _agg_kernel(y_hbm, src_hbm, dst_hbm, parts_out,
                sgrp, dgrp, rows0, rows1, obuf, sem0, sem1, acc):
    cid = lax.axis_index("c")
    sid = lax.axis_index("s")
    wid = cid * 16 + sid
    z16 = jnp.zeros((16,), jnp.float32)

    @pl.loop(0, _OB)
    def _(r):
        for c in range(_D // 16):
            obuf[r, pl.ds(c * 16, 16)] = z16

    @pl.loop(0, _RPT // _OB)
    def _(k):
        pltpu.sync_copy(obuf, acc.at[pl.ds(sid * _RPT + k * _OB, _OB)])

    plsc.subcore_barrier()

    @pl.loop(0, _NGRP)
    def _(g):
        pltpu.sync_copy(src_hbm.at[wid, pl.ds(g * _GRP, _GRP)], sgrp)
        pltpu.sync_copy(dst_hbm.at[wid, pl.ds(g * _GRP, _GRP)], dgrp)
        pltpu.async_copy(y_hbm.at[sgrp.at[0]], rows0, sem0)

        @pl.loop(0, _GRP // 2)
        def _(t):
            j = t * 2
            pltpu.make_async_copy(y_hbm.at[sgrp.at[j]], rows0, sem0).wait()
            pltpu.async_copy(y_hbm.at[sgrp.at[j + 1]], rows1, sem1)
            pltpu.sync_copy(rows0, acc.at[dgrp.at[j]], add=True)
            pltpu.make_async_copy(y_hbm.at[sgrp.at[j + 1]], rows1, sem1).wait()
            jn = (j + 2) % _GRP
            pltpu.async_copy(y_hbm.at[sgrp.at[jn]], rows0, sem0)
            pltpu.sync_copy(rows1, acc.at[dgrp.at[j + 1]], add=True)

        pltpu.make_async_copy(y_hbm.at[sgrp.at[0]], rows0, sem0).wait()

    plsc.subcore_barrier()

    @pl.loop(0, _RPT // _OB)
    def _(k):
        pltpu.sync_copy(acc.at[pl.ds(sid * _RPT + k * _OB, _OB)], obuf)
        pltpu.sync_copy(
            obuf,
            parts_out.at[pl.ds(cid * _NPAD + sid * _RPT + k * _OB, _OB)])


def _dinv_body(dp_ref, out_ref):
    p = dp_ref[...]
    deg = p[0:80, :] + p[80:160, :] + 1.0
    out_ref[...] = lax.rsqrt(deg)


_dinv_call = pl.pallas_call(
    _dinv_body,
    out_shape=jax.ShapeDtypeStruct((80, 128), jnp.float32),
)


def _y1_body(x_ref, dinv_ref, w_ref, out_ref):
    xw = jnp.dot(x_ref[...], w_ref[...],
                 preferred_element_type=jnp.float32,
                 precision=lax.Precision.HIGHEST)
    out_ref[...] = dinv_ref[...] * xw


_y1_call = pl.pallas_call(
    _y1_body,
    grid=(_NBLK,),
    in_specs=[
        pl.BlockSpec((_BLK, _D), lambda i: (i, 0)),
        pl.BlockSpec((_BLK, 1), lambda i: (i, 0)),
        pl.BlockSpec((_D, _H), lambda i: (0, 0)),
    ],
    out_specs=pl.BlockSpec((_BLK, _H), lambda i: (i, 0)),
    out_shape=jax.ShapeDtypeStruct((_NPAD, _H), jnp.float32),
)


def _layer_body(pa_ref, pb_ref, y_ref, dinv_ref, b_ref, w_ref, out_ref):
    dinv = dinv_ref[...]
    h = dinv * (pa_ref[...] + pb_ref[...] + y_ref[...]) + b_ref[...]
    h = jnp.maximum(h, 0.0)
    out_ref[...] = dinv * jnp.dot(h, w_ref[...],
                                  preferred_element_type=jnp.float32,
                                  precision=lax.Precision.HIGHEST)


_layer_call = pl.pallas_call(
    _layer_body,
    grid=(_NBLK,),
    in_specs=[
        pl.BlockSpec((_BLK, _H), lambda i: (i, 0)),
        pl.BlockSpec((_BLK, _H), lambda i: (i + _NBLK, 0)),
        pl.BlockSpec((_BLK, _H), lambda i: (i, 0)),
        pl.BlockSpec((_BLK, 1), lambda i: (i, 0)),
        pl.BlockSpec((1, _H), lambda i: (0, 0)),
        pl.BlockSpec((_H, _H), lambda i: (0, 0)),
    ],
    out_specs=pl.BlockSpec((_BLK, _H), lambda i: (i, 0)),
    out_shape=jax.ShapeDtypeStruct((_NPAD, _H), jnp.float32),
)


def _final_body(pa_ref, pb_ref, y_ref, dinv_ref, b_ref, wh_ref,
                txt_ref, wt_ref, bc_ref, out_ref):
    dinv = dinv_ref[...]
    h2 = dinv * (pa_ref[...] + pb_ref[...] + y_ref[...]) + b_ref[...]
    const = jnp.dot(txt_ref[...], wt_ref[...],
                    preferred_element_type=jnp.float32,
                    precision=lax.Precision.HIGHEST)
    out_ref[...] = (jnp.dot(h2, wh_ref[...],
                            preferred_element_type=jnp.float32,
                            precision=lax.Precision.HIGHEST)
                    + const + bc_ref[...])


_final_call = pl.pallas_call(
    _final_body,
    grid=(_NBLK,),
    in_specs=[
        pl.BlockSpec((_BLK, _H), lambda i: (i, 0)),
        pl.BlockSpec((_BLK, _H), lambda i: (i + _NBLK, 0)),
        pl.BlockSpec((_BLK, _H), lambda i: (i, 0)),
        pl.BlockSpec((_BLK, 1), lambda i: (i, 0)),
        pl.BlockSpec((1, _H), lambda i: (0, 0)),
        pl.BlockSpec((_H, _OUT), lambda i: (0, 0)),
        pl.BlockSpec((1, _TD), lambda i: (0, 0)),
        pl.BlockSpec((_TD, _OUT), lambda i: (0, 0)),
        pl.BlockSpec((1, _OUT), lambda i: (0, 0)),
    ],
    out_specs=pl.BlockSpec((_BLK, _OUT), lambda i: (i, 0)),
    out_shape=jax.ShapeDtypeStruct((_NPAD, _OUT), jnp.float32),
)


def kernel(x, edge_index, text_vec, W1, b1, W2, b2, Wc, bc):
    src = edge_index[0]
    dst = edge_index[1]
    pad = _EPAD - _E
    fill = _N + (jnp.arange(pad, dtype=jnp.int32) % (_NPAD - _N))
    src_p = jnp.concatenate([src, fill]).reshape(_NW, _NCH, _CHUNK)
    dst_p = jnp.concatenate([dst, fill]).reshape(_NW, _NCH, _CHUNK)

    deg_parts = _deg_kernel(dst_p)
    dinv2d = _dinv_call(deg_parts.reshape(160, 128))
    dinv_col = dinv2d.reshape(_NPAD, 1)

    x_p = jnp.pad(x, ((0, _NPAD - _N), (0, 0)))
    y1 = _y1_call(x_p, dinv_col, W1)
    p1 = _agg_kernel(y1, src_p, dst_p)
    y2 = _layer_call(p1, p1, y1, dinv_col, b1.reshape(1, _H), W2)
    p2 = _agg_kernel(y2, src_p, dst_p)
    logits = _final_call(p2, p2, y2, dinv_col, b2.reshape(1, _H),
                         Wc[:, :_H].T, text_vec.reshape(1, _TD),
                         Wc[:, _H:].T, bc.reshape(1, _OUT))
    return logits[:_N]


# speedup vs baseline: 25.4445x; 1.0156x over previous
import functools

import jax
import jax.numpy as jnp
from jax import lax
from jax.experimental import pallas as pl
from jax.experimental.pallas import tpu as pltpu
from jax.experimental.pallas import tpu_sc as plsc

_N = 10000
_E = 320000
_D = 128
_H = 128
_TD = 768
_OUT = 64

_NPAD = 10240
_NDUMMY = _NPAD - 1
_NW = 32
_CHUNK = 128
_NCH = 80
_EPT = _NCH * _CHUNK
_EPAD = _EPT * _NW
_RPT = _NPAD // 16

_BLK = 512
_NBLK = _NPAD // _BLK


def _sc_mesh():
    return plsc.VectorSubcoreMesh(core_axis_name="c", subcore_axis_name="s")


@functools.partial(
    pl.kernel,
    out_type=jax.ShapeDtypeStruct((2, _NPAD), jnp.float32),
    mesh=_sc_mesh(),
    scratch_types=[
        pltpu.VMEM((_NCH, _CHUNK), jnp.int32),
        pltpu.VMEM((_CHUNK,), jnp.float32),
        pltpu.VMEM((_RPT,), jnp.float32),
        pltpu.VMEM((_NPAD,), jnp.float32),
        pltpu.VMEM_SHARED((_NPAD,), jnp.float32),
    ],
)
def _deg_kernel(dst_hbm, deg_out, dstbuf, ones, zbuf, degbuf, sc_deg):
    cid = lax.axis_index("c")
    sid = lax.axis_index("s")
    wid = cid * 16 + sid
    z16 = jnp.zeros((16,), jnp.float32)
    o16 = jnp.ones((16,), jnp.float32)

    @pl.loop(0, _RPT // 16)
    def _(i):
        zbuf[pl.ds(i * 16, 16)] = z16

    for i in range(_CHUNK // 16):
        ones[pl.ds(i * 16, 16)] = o16

    pltpu.sync_copy(zbuf, sc_deg.at[pl.ds(sid * _RPT, _RPT)])
    pltpu.sync_copy(dst_hbm.at[wid], dstbuf)
    plsc.subcore_barrier()

    @pl.loop(0, _NCH)
    def _(j):
        pltpu.sync_copy(ones, sc_deg.at[dstbuf.at[j]], add=True)

    plsc.subcore_barrier()

    @pl.when(sid == 0)
    def _():
        pltpu.sync_copy(sc_deg, degbuf)
        pltpu.sync_copy(degbuf, deg_out.at[cid])


_GRP = 16
_NGRP = _NCH // _GRP
_OB = 32
_NOB = _RPT // _OB


@functools.partial(
    pl.kernel,
    out_type=jax.ShapeDtypeStruct((2 * _NPAD, _D), jnp.float32),
    mesh=_sc_mesh(),
    scratch_types=[
        pltpu.VMEM((_GRP, _CHUNK), jnp.int32),
        pltpu.VMEM((_GRP, _CHUNK), jnp.int32),
        pltpu.VMEM((_CHUNK, _D), jnp.float32),
        pltpu.VMEM((_CHUNK, _D), jnp.float32),
        pltpu.VMEM((_OB, _D), jnp.float32),
        pltpu.VMEM((_OB, _D), jnp.float32),
        pltpu.SemaphoreType.DMA,
        pltpu.SemaphoreType.DMA,
        pltpu.SemaphoreType.DMA,
        pltpu.SemaphoreType.DMA,
        pltpu.VMEM_SHARED((_NPAD, _D), jnp.float32),
    ],
)
def _agg_kernel(y_hbm, src_hbm, dst_hbm, parts_out,
                sgrp, dgrp, rows0, rows1, obufa, obufb,
                sem0, sem1, zsem, wsem, acc):
    cid = lax.axis_index("c")
    sid = lax.axis_index("s")
    wid = cid * 16 + sid
    z16 = jnp.zeros((16,), jnp.float32)

    @pl.loop(0, _OB)
    def _(r):
        for c in range(_D // 16):
            obufa[r, pl.ds(c * 16, 16)] = z16

    @pl.loop(0, _NOB)
    def _(k):
        pltpu.async_copy(obufa, acc.at[pl.ds(sid * _RPT + k * _OB, _OB)],
                         zsem)

    pltpu.sync_copy(src_hbm.at[wid, pl.ds(0, _GRP)], sgrp)
    pltpu.sync_copy(dst_hbm.at[wid, pl.ds(0, _GRP)], dgrp)
    pltpu.async_copy(y_hbm.at[sgrp.at[0]], rows0, sem0)

    @pl.loop(0, _NOB)
    def _(k):
        pltpu.make_async_copy(
            obufa, acc.at[pl.ds(sid * _RPT + k * _OB, _OB)], zsem).wait()

    plsc.subcore_barrier()

    @pl.loop(0, _NGRP)
    def _(g):
        @pl.when(g > 0)
        def _():
            pltpu.sync_copy(src_hbm.at[wid, pl.ds(g * _GRP, _GRP)], sgrp)
            pltpu.sync_copy(dst_hbm.at[wid, pl.ds(g * _GRP, _GRP)], dgrp)
            pltpu.async_copy(y_hbm.at[sgrp.at[0]], rows0, sem0)

        @pl.loop(0, _GRP // 2)
        def _(t):
            j = t * 2
            pltpu.make_async_copy(y_hbm.at[sgrp.at[j]], rows0, sem0).wait()
            pltpu.async_copy(y_hbm.at[sgrp.at[j + 1]], rows1, sem1)
            pltpu.sync_copy(rows0, acc.at[dgrp.at[j]], add=True)
            pltpu.make_async_copy(y_hbm.at[sgrp.at[j + 1]], rows1, sem1).wait()
            jn = (j + 2) % _GRP
            pltpu.async_copy(y_hbm.at[sgrp.at[jn]], rows0, sem0)
            pltpu.sync_copy(rows1, acc.at[dgrp.at[j + 1]], add=True)

        pltpu.make_async_copy(y_hbm.at[sgrp.at[0]], rows0, sem0).wait()

    plsc.subcore_barrier()

    @pl.loop(0, _NOB // 2)
    def _(k):
        sa = sid * _RPT + 2 * k * _OB
        sb = sa + _OB
        pltpu.sync_copy(acc.at[pl.ds(sa, _OB)], obufa)
        pltpu.async_copy(obufa, parts_out.at[pl.ds(cid * _NPAD + sa, _OB)],
                         wsem)
        pltpu.sync_copy(acc.at[pl.ds(sb, _OB)], obufb)
        pltpu.async_copy(obufb, parts_out.at[pl.ds(cid * _NPAD + sb, _OB)],
                         wsem)
        pltpu.make_async_copy(
            obufa, parts_out.at[pl.ds(cid * _NPAD + sa, _OB)], wsem).wait()
        pltpu.make_async_copy(
            obufb, parts_out.at[pl.ds(cid * _NPAD + sb, _OB)], wsem).wait()


def _dinv(da_ref, db_ref):
    return lax.rsqrt(da_ref[...] + db_ref[...] + 1.0)


def _y1_body(x_ref, da_ref, db_ref, w_ref, out_ref):
    xw = jnp.dot(x_ref[...], w_ref[...],
                 preferred_element_type=jnp.float32,
                 precision=lax.Precision.HIGHEST)
    out_ref[...] = _dinv(da_ref, db_ref) * xw


_y1_call = pl.pallas_call(
    _y1_body,
    grid=(_NBLK,),
    in_specs=[
        pl.BlockSpec((_BLK, _D), lambda i: (i, 0)),
        pl.BlockSpec((_BLK, 1), lambda i: (i, 0)),
        pl.BlockSpec((_BLK, 1), lambda i: (i + _NBLK, 0)),
        pl.BlockSpec((_D, _H), lambda i: (0, 0)),
    ],
    out_specs=pl.BlockSpec((_BLK, _H), lambda i: (i, 0)),
    out_shape=jax.ShapeDtypeStruct((_NPAD, _H), jnp.float32),
)


def _layer_body(pa_ref, pb_ref, y_ref, da_ref, db_ref, b_ref, w_ref, out_ref):
    dinv = _dinv(da_ref, db_ref)
    h = dinv * (pa_ref[...] + pb_ref[...] + y_ref[...]) + b_ref[...]
    h = jnp.maximum(h, 0.0)
    out_ref[...] = dinv * jnp.dot(h, w_ref[...],
                                  preferred_element_type=jnp.float32,
                                  precision=lax.Precision.HIGHEST)


_layer_call = pl.pallas_call(
    _layer_body,
    grid=(_NBLK,),
    in_specs=[
        pl.BlockSpec((_BLK, _H), lambda i: (i, 0)),
        pl.BlockSpec((_BLK, _H), lambda i: (i + _NBLK, 0)),
        pl.BlockSpec((_BLK, _H), lambda i: (i, 0)),
        pl.BlockSpec((_BLK, 1), lambda i: (i, 0)),
        pl.BlockSpec((_BLK, 1), lambda i: (i + _NBLK, 0)),
        pl.BlockSpec((1, _H), lambda i: (0, 0)),
        pl.BlockSpec((_H, _H), lambda i: (0, 0)),
    ],
    out_specs=pl.BlockSpec((_BLK, _H), lambda i: (i, 0)),
    out_shape=jax.ShapeDtypeStruct((_NPAD, _H), jnp.float32),
)


def _final_body(pa_ref, pb_ref, y_ref, da_ref, db_ref, b_ref, wh_ref,
                txt_ref, wt_ref, bc_ref, out_ref):
    dinv = _dinv(da_ref, db_ref)
    h2 = dinv * (pa_ref[...] + pb_ref[...] + y_ref[...]) + b_ref[...]
    const = jnp.dot(txt_ref[...], wt_ref[...],
                    preferred_element_type=jnp.float32,
                    precision=lax.Precision.HIGHEST)
    out_ref[...] = (jnp.dot(h2, wh_ref[...],
                            preferred_element_type=jnp.float32,
                            precision=lax.Precision.HIGHEST)
                    + const + bc_ref[...])


_final_call = pl.pallas_call(
    _final_body,
    grid=(_NBLK,),
    in_specs=[
        pl.BlockSpec((_BLK, _H), lambda i: (i, 0)),
        pl.BlockSpec((_BLK, _H), lambda i: (i + _NBLK, 0)),
        pl.BlockSpec((_BLK, _H), lambda i: (i, 0)),
        pl.BlockSpec((_BLK, 1), lambda i: (i, 0)),
        pl.BlockSpec((_BLK, 1), lambda i: (i + _NBLK, 0)),
        pl.BlockSpec((1, _H), lambda i: (0, 0)),
        pl.BlockSpec((_H, _OUT), lambda i: (0, 0)),
        pl.BlockSpec((1, _TD), lambda i: (0, 0)),
        pl.BlockSpec((_TD, _OUT), lambda i: (0, 0)),
        pl.BlockSpec((1, _OUT), lambda i: (0, 0)),
    ],
    out_specs=pl.BlockSpec((_BLK, _OUT), lambda i: (i, 0)),
    out_shape=jax.ShapeDtypeStruct((_NPAD, _OUT), jnp.float32),
)


def kernel(x, edge_index, text_vec, W1, b1, W2, b2, Wc, bc):
    src = edge_index[0]
    dst = edge_index[1]
    pad = _EPAD - _E
    fill = _N + (jnp.arange(pad, dtype=jnp.int32) % (_NPAD - _N))
    src_p = jnp.concatenate([src, fill]).reshape(_NW, _NCH, _CHUNK)
    dst_p = jnp.concatenate([dst, fill]).reshape(_NW, _NCH, _CHUNK)

    deg_parts = _deg_kernel(dst_p)
    deg_col = deg_parts.reshape(2 * _NPAD, 1)

    x_p = jnp.pad(x, ((0, _NPAD - _N), (0, 0)))
    y1 = _y1_call(x_p, deg_col, deg_col, W1)
    p1 = _agg_kernel(y1, src_p, dst_p)
    y2 = _layer_call(p1, p1, y1, deg_col, deg_col, b1.reshape(1, _H), W2)
    p2 = _agg_kernel(y2, src_p, dst_p)
    logits = _final_call(p2, p2, y2, deg_col, deg_col, b2.reshape(1, _H),
                         Wc[:, :_H].T, text_vec.reshape(1, _TD),
                         Wc[:, _H:].T, bc.reshape(1, _OUT))
    return logits[:_N]
